# baseline (speedup 1.0000x reference)
import functools
import jax
import jax.numpy as jnp
from jax import lax
from jax.experimental import pallas as pl
from jax.experimental.pallas import tpu as pltpu
from jax.experimental.pallas import tpu_sc as plsc

KNN = 20
CPAD = 128
F32_MIN = jnp.finfo(jnp.float32).min


def _knn_body(n_pts, xr_ref, xa_ref, idx_ref):
    xr = xr_ref[0]
    xa = xa_ref[0]
    g = lax.dot_general(xr, xa, (((1,), (1,)), ((), ())),
                        preferred_element_type=jnp.float32)
    inner = -2.0 * g
    s = (-jnp.sum(xr * xr, axis=1)[:, None] - inner) \
        - jnp.sum(xa * xa, axis=1)[None, :]
    iota = lax.broadcasted_iota(jnp.int32, s.shape, 1)
    sels = []
    for _ in range(KNN):
        m = jnp.max(s, axis=1, keepdims=True)
        sel = jnp.min(jnp.where(s >= m, iota, n_pts), axis=1)
        sels.append(sel)
        s = jnp.where(iota == sel[:, None], F32_MIN, s)
    idx_ref[0] = jnp.stack(sels, axis=1)


def _knn(x, row_block=1024):
    b, n, c = x.shape
    return pl.pallas_call(
        functools.partial(_knn_body, n),
        grid=(b, n // row_block),
        in_specs=[
            pl.BlockSpec((1, row_block, c), lambda bi, j: (bi, j, 0)),
            pl.BlockSpec((1, n, c), lambda bi, j: (bi, 0, 0)),
        ],
        out_specs=pl.BlockSpec((1, row_block, KNN), lambda bi, j: (bi, j, 0)),
        out_shape=jax.ShapeDtypeStruct((b, n, KNN), jnp.int32),
        compiler_params=pltpu.CompilerParams(
            dimension_semantics=("arbitrary", "arbitrary")),
    )(x, x)


def _sc_gather(xf, idxf, n_pts, chunk=16):
    m_rows, c = xf.shape
    info = plsc.get_sparse_core_info()
    nc, ns = info.num_cores, info.num_subcores
    nw = nc * ns
    rw = m_rows // nw
    assert rw % chunk == 0 and n_pts % rw == 0
    nidx = chunk * KNN
    sub = []
    off = 0
    while off < nidx:
        ln = min(128, nidx - off)
        sub.append((off, ln))
        off += ln

    mesh = plsc.VectorSubcoreMesh(core_axis_name="c", subcore_axis_name="s")

    @functools.partial(
        pl.kernel, mesh=mesh,
        out_type=jax.ShapeDtypeStruct((m_rows * KNN, c), jnp.float32),
        scratch_types=[
            pltpu.VMEM((nidx,), jnp.int32),
            pltpu.VMEM((nidx, c), jnp.float32),
            pltpu.VMEM((nidx, c), jnp.float32),
            pltpu.SemaphoreType.DMA,
            pltpu.SemaphoreType.DMA,
        ],
    )
    def k(x_hbm, idx_hbm, xg_hbm, idxb, rows0, rows1, gsem, wsem):
        wid = lax.axis_index("s") * nc + lax.axis_index("c")
        base = wid * rw
        boff = (base // n_pts) * n_pts

        def fetch(ci, rows):
            rbase = base + ci * chunk
            pltpu.sync_copy(idx_hbm.at[pl.ds(rbase * KNN, nidx)], idxb)
            for j in range(nidx // 16):
                sl = pl.ds(j * 16, 16)
                idxb[sl] = idxb[sl] + boff
            return [
                pltpu.async_copy(x_hbm.at[idxb.at[pl.ds(soff, slen)]],
                                 rows.at[pl.ds(soff, slen)], gsem)
                for soff, slen in sub
            ]

        nch = rw // chunk
        bufs = (rows0, rows1)
        hs = fetch(0, rows0)
        for ci in range(nch):
            for h in hs:
                h.wait()
            cur = bufs[ci % 2]
            if ci + 1 < nch:
                hs = fetch(ci + 1, bufs[(ci + 1) % 2])
            rbase = base + ci * chunk
            pltpu.async_copy(
                cur, xg_hbm.at[pl.ds(rbase * KNN, nidx)], wsem).wait()

    return k(xf, idxf)


def _edge_body(c_dim, w_ref, xg_ref, xi_ref, mx_ref, s1_ref, s2_ref):
    bi = pl.program_id(0)
    j = pl.program_id(1)
    w = w_ref[...]
    xi = xi_ref[0][:, :c_dim]
    mx = None
    a1 = None
    a2 = None
    for kk in range(KNN):
        xj = xg_ref[0][:, kk * CPAD:kk * CPAD + c_dim]
        f = jnp.concatenate([xj - xi, xi], axis=1)
        y = lax.dot_general(f, w, (((1,), (1,)), ((), ())),
                            preferred_element_type=jnp.float32)
        mx = y if mx is None else jnp.maximum(mx, y)
        a1 = y if a1 is None else a1 + y
        a2 = y * y if a2 is None else a2 + y * y
    mx_ref[0] = mx
    ls1 = jnp.sum(a1, axis=0)[None, :]
    ls2 = jnp.sum(a2, axis=0)[None, :]

    @pl.when((j == 0) & (bi == 0))
    def _():
        s1_ref[...] = ls1
        s2_ref[...] = ls2

    @pl.when((j != 0) | (bi != 0))
    def _():
        s1_ref[...] = s1_ref[...] + ls1
        s2_ref[...] = s2_ref[...] + ls2


def _edge(xg, x, w, c_dim, row_block=512):
    b, n, _ = x.shape
    o = w.shape[0]
    return pl.pallas_call(
        functools.partial(_edge_body, c_dim),
        grid=(b, n // row_block),
        in_specs=[
            pl.BlockSpec((o, 2 * c_dim), lambda bi, j: (0, 0)),
            pl.BlockSpec((1, row_block, KNN * CPAD), lambda bi, j: (bi, j, 0)),
            pl.BlockSpec((1, row_block, CPAD), lambda bi, j: (bi, j, 0)),
        ],
        out_specs=[
            pl.BlockSpec((1, row_block, o), lambda bi, j: (bi, j, 0)),
            pl.BlockSpec((1, o), lambda bi, j: (0, 0)),
            pl.BlockSpec((1, o), lambda bi, j: (0, 0)),
        ],
        out_shape=[
            jax.ShapeDtypeStruct((b, n, o), jnp.float32),
            jax.ShapeDtypeStruct((1, o), jnp.float32),
            jax.ShapeDtypeStruct((1, o), jnp.float32),
        ],
        compiler_params=pltpu.CompilerParams(
            dimension_semantics=("arbitrary", "arbitrary")),
    )(w, xg.reshape(b, n, KNN * CPAD), x)


def _affine_lrelu_body(o_dim, x_ref, mu_ref, sg_ref, g_ref, b_ref, o_ref):
    z = (x_ref[0] - mu_ref[...]) / sg_ref[...] * g_ref[...] + b_ref[...]
    z = jnp.where(z > 0, z, 0.2 * z)
    if o_dim < CPAD:
        z = jnp.concatenate(
            [z, jnp.zeros((z.shape[0], CPAD - o_dim), jnp.float32)], axis=1)
    o_ref[0] = z


def _affine_lrelu_pad(x, mean, sig, g, beta, row_block=512):
    b, n, o = x.shape
    vec = lambda v: v.reshape(1, o)
    return pl.pallas_call(
        functools.partial(_affine_lrelu_body, o),
        grid=(b, n // row_block),
        in_specs=[pl.BlockSpec((1, row_block, o), lambda bi, j: (bi, j, 0))]
        + [pl.BlockSpec((1, o), lambda bi, j: (0, 0))] * 4,
        out_specs=pl.BlockSpec((1, row_block, max(o, CPAD)),
                               lambda bi, j: (bi, j, 0)),
        out_shape=jax.ShapeDtypeStruct((b, n, max(o, CPAD)), jnp.float32),
        compiler_params=pltpu.CompilerParams(
            dimension_semantics=("arbitrary", "arbitrary")),
    )(x, vec(mean), vec(sig), vec(g), vec(beta))


def _head5_body(w5_ref, w6b_ref, cat_ref, y5max_ref, s15_ref, s25_ref,
                r6_ref, rs1_ref, rs2_ref):
    bi = pl.program_id(0)
    j = pl.program_id(1)
    cat = cat_ref[0]
    y5 = lax.dot_general(cat, w5_ref[...], (((1,), (1,)), ((), ())),
                         preferred_element_type=jnp.float32)
    r6 = lax.dot_general(cat, w6b_ref[...], (((1,), (1,)), ((), ())),
                         preferred_element_type=jnp.float32)
    r6_ref[0] = r6
    bm = jnp.max(y5, axis=0)[None, None, :]
    ls1 = jnp.sum(y5, axis=0)[None, :]
    ls2 = jnp.sum(y5 * y5, axis=0)[None, :]
    lr1 = jnp.sum(r6, axis=0)[None, None, :]
    lr2 = jnp.sum(r6 * r6, axis=0)[None, None, :]

    @pl.when(j == 0)
    def _():
        y5max_ref[...] = bm
        rs1_ref[...] = lr1
        rs2_ref[...] = lr2

    @pl.when(j != 0)
    def _():
        y5max_ref[...] = jnp.maximum(y5max_ref[...], bm)
        rs1_ref[...] = rs1_ref[...] + lr1
        rs2_ref[...] = rs2_ref[...] + lr2

    @pl.when((j == 0) & (bi == 0))
    def _():
        s15_ref[...] = ls1
        s25_ref[...] = ls2

    @pl.when((j != 0) | (bi != 0))
    def _():
        s15_ref[...] = s15_ref[...] + ls1
        s25_ref[...] = s25_ref[...] + ls2


def _head5(cat, w5, w6b, row_block=512):
    b, n, ci = cat.shape
    o5 = w5.shape[0]
    o6 = w6b.shape[0]
    return pl.pallas_call(
        _head5_body,
        grid=(b, n // row_block),
        in_specs=[
            pl.BlockSpec((o5, ci), lambda bi, j: (0, 0)),
            pl.BlockSpec((o6, ci), lambda bi, j: (0, 0)),
            pl.BlockSpec((1, row_block, ci), lambda bi, j: (bi, j, 0)),
        ],
        out_specs=[
            pl.BlockSpec((1, 1, o5), lambda bi, j: (bi, 0, 0)),
            pl.BlockSpec((1, o5), lambda bi, j: (0, 0)),
            pl.BlockSpec((1, o5), lambda bi, j: (0, 0)),
            pl.BlockSpec((1, row_block, o6), lambda bi, j: (bi, j, 0)),
            pl.BlockSpec((1, 1, o6), lambda bi, j: (bi, 0, 0)),
            pl.BlockSpec((1, 1, o6), lambda bi, j: (bi, 0, 0)),
        ],
        out_shape=[
            jax.ShapeDtypeStruct((b, 1, o5), jnp.float32),
            jax.ShapeDtypeStruct((1, o5), jnp.float32),
            jax.ShapeDtypeStruct((1, o5), jnp.float32),
            jax.ShapeDtypeStruct((b, n, o6), jnp.float32),
            jax.ShapeDtypeStruct((b, 1, o6), jnp.float32),
            jax.ShapeDtypeStruct((b, 1, o6), jnp.float32),
        ],
        compiler_params=pltpu.CompilerParams(
            dimension_semantics=("arbitrary", "arbitrary")),
    )(w5, w6b, cat)


def _bias6_body(g_ref, w_ref, o_ref):
    o_ref[...] = lax.dot_general(g_ref[...], w_ref[...],
                                 (((1,), (1,)), ((), ())),
                                 preferred_element_type=jnp.float32)


def _bias6(gmax, w6a):
    b, _ = gmax.shape
    o = w6a.shape[0]
    return pl.pallas_call(
        _bias6_body,
        out_shape=jax.ShapeDtypeStruct((b, o), jnp.float32),
    )(gmax, w6a)


def _head7_body(w7_ref, mu_ref, sg_ref, g_ref, be_ref, r6_ref, bias_ref,
                y7max_ref, s17_ref, s27_ref):
    bi = pl.program_id(0)
    j = pl.program_id(1)
    y6 = r6_ref[0] + bias_ref[0]
    z = (y6 - mu_ref[...]) / sg_ref[...] * g_ref[...] + be_ref[...]
    h = jnp.where(z > 0, z, 0.2 * z)
    y7 = lax.dot_general(h, w7_ref[...], (((1,), (1,)), ((), ())),
                         preferred_element_type=jnp.float32)
    bm = jnp.max(y7, axis=0)[None, None, :]
    ls1 = jnp.sum(y7, axis=0)[None, :]
    ls2 = jnp.sum(y7 * y7, axis=0)[None, :]

    @pl.when(j == 0)
    def _():
        y7max_ref[...] = bm

    @pl.when(j != 0)
    def _():
        y7max_ref[...] = jnp.maximum(y7max_ref[...], bm)

    @pl.when((j == 0) & (bi == 0))
    def _():
        s17_ref[...] = ls1
        s27_ref[...] = ls2

    @pl.when((j != 0) | (bi != 0))
    def _():
        s17_ref[...] = s17_ref[...] + ls1
        s27_ref[...] = s27_ref[...] + ls2


def _head7(r6, bias, mean, sig, g6, b6, w7, row_block=512):
    b, n, ci = r6.shape
    o = w7.shape[0]
    vec = lambda v: v.reshape(1, ci)
    return pl.pallas_call(
        _head7_body,
        grid=(b, n // row_block),
        in_specs=[pl.BlockSpec((o, ci), lambda bi, j: (0, 0))]
        + [pl.BlockSpec((1, ci), lambda bi, j: (0, 0))] * 4
        + [
            pl.BlockSpec((1, row_block, ci), lambda bi, j: (bi, j, 0)),
            pl.BlockSpec((1, 1, ci), lambda bi, j: (bi, 0, 0)),
        ],
        out_specs=[
            pl.BlockSpec((1, 1, o), lambda bi, j: (bi, 0, 0)),
            pl.BlockSpec((1, o), lambda bi, j: (0, 0)),
            pl.BlockSpec((1, o), lambda bi, j: (0, 0)),
        ],
        out_shape=[
            jax.ShapeDtypeStruct((b, 1, o), jnp.float32),
            jax.ShapeDtypeStruct((1, o), jnp.float32),
            jax.ShapeDtypeStruct((1, o), jnp.float32),
        ],
        compiler_params=pltpu.CompilerParams(
            dimension_semantics=("arbitrary", "arbitrary")),
    )(w7, vec(mean), vec(sig), vec(g6), vec(b6), r6,
      bias.reshape(b, 1, ci))


def _bn_stats(s1, s2, count):
    mean = s1 / count
    var = s2 / count - mean * mean
    return mean, jnp.sqrt(var + 1e-5)


def _lrelu(z):
    return jnp.where(z > 0, z, 0.2 * z)


def _edgeconv(x, c_dim, w, g, bb):
    b, n, _ = x.shape
    grp = 2
    mxs, s1s, s2s = [], [], []
    for bi in range(0, b, grp):
        xb = lax.slice_in_dim(x, bi, bi + grp, axis=0)
        idx = _knn(xb)
        xg = _sc_gather(xb.reshape(grp * n, CPAD),
                        idx.reshape(grp * n * KNN), n)
        mxb, s1b, s2b = _edge(xg, xb, w, c_dim)
        mxs.append(mxb)
        s1s.append(s1b)
        s2s.append(s2b)
    s1 = sum(v[0] for v in s1s)
    s2 = sum(v[0] for v in s2s)
    mx = jnp.concatenate(mxs, axis=0)
    mean, sig = _bn_stats(s1, s2, b * n * KNN)
    return _affine_lrelu_pad(mx, mean, sig, g, bb)


def kernel(x, W1, W2, W3, W4, W5, W6, W7,
           g1, g2, g3, g4, g5, g6, g7,
           b1, b2, b3, b4, b5, b6, b7):
    b, n, c0 = x.shape
    xp = jnp.pad(x, ((0, 0), (0, 0), (0, CPAD - c0)))
    x1 = _edgeconv(xp, c0, W1, g1, b1)
    x2 = _edgeconv(x1, 64, W2, g2, b2)
    x3 = _edgeconv(x2, 64, W3, g3, b3)
    x4 = _edgeconv(x3, 128, W4, g4, b4)
    cat = jnp.concatenate([x1[..., :64], x2[..., :64], x3, x4], axis=-1)

    w6a, w6b = W6[:, :W5.shape[0]], W6[:, W5.shape[0]:]
    y5max, s15, s25, r6, rs1, rs2 = _head5(cat, W5, w6b)
    y5max, rs1, rs2 = y5max[:, 0], rs1[:, 0], rs2[:, 0]
    p = b * n
    mean5, sig5 = _bn_stats(s15[0], s25[0], p)
    gmax = _lrelu((y5max - mean5) / sig5 * g5 + b5)
    bias = _bias6(gmax, w6a)

    mean6 = jnp.sum(rs1 + n * bias, axis=0) / p
    e2 = jnp.sum(rs2 + 2.0 * bias * rs1 + n * bias * bias, axis=0) / p
    var6 = e2 - mean6 * mean6
    sig6 = jnp.sqrt(var6 + 1e-5)

    y7max, s17, s27 = _head7(r6, bias, mean6, sig6, g6, b6, W7)
    mean7, sig7 = _bn_stats(s17[0], s27[0], p)
    return _lrelu((y7max[:, 0] - mean7) / sig7 * g7 + b7)


# speedup vs baseline: 1.2076x; 1.2076x over previous
import functools
import jax
import jax.numpy as jnp
from jax import lax
from jax.experimental import pallas as pl
from jax.experimental.pallas import tpu as pltpu
from jax.experimental.pallas import tpu_sc as plsc

KNN = 20
CPAD = 128
F32_MIN = jnp.finfo(jnp.float32).min


def _knn_body(n_pts, xr_ref, xa_ref, idx_ref):
    xr = xr_ref[0]
    xa = xa_ref[0]
    g = lax.dot_general(xr, xa, (((1,), (1,)), ((), ())),
                        preferred_element_type=jnp.float32)
    inner = -2.0 * g
    s = (-jnp.sum(xr * xr, axis=1)[:, None] - inner) \
        - jnp.sum(xa * xa, axis=1)[None, :]
    iota = lax.broadcasted_iota(jnp.int32, s.shape, 1)
    sels = []
    for _ in range(KNN):
        m = jnp.max(s, axis=1, keepdims=True)
        sel = jnp.min(jnp.where(s >= m, iota, n_pts), axis=1)
        sels.append(sel)
        s = jnp.where(iota == sel[:, None], F32_MIN, s)
    idx_ref[0] = jnp.stack(sels, axis=1)


def _knn(x, row_block=512):
    b, n, c = x.shape
    return pl.pallas_call(
        functools.partial(_knn_body, n),
        grid=(b, n // row_block),
        in_specs=[
            pl.BlockSpec((1, row_block, c), lambda bi, j: (bi, j, 0)),
            pl.BlockSpec((1, n, c), lambda bi, j: (bi, 0, 0)),
        ],
        out_specs=pl.BlockSpec((1, row_block, KNN), lambda bi, j: (bi, j, 0)),
        out_shape=jax.ShapeDtypeStruct((b, n, KNN), jnp.int32),
        compiler_params=pltpu.CompilerParams(
            dimension_semantics=("arbitrary", "arbitrary")),
    )(x, x)


def _sc_gather(xf, idxf, n_pts, chunk=16):
    m_rows, c = xf.shape
    info = plsc.get_sparse_core_info()
    nc, ns = info.num_cores, info.num_subcores
    nw = nc * ns
    rw = m_rows // nw
    assert rw % chunk == 0 and n_pts % rw == 0
    nidx = chunk * KNN
    sub = []
    off = 0
    while off < nidx:
        ln = min(128, nidx - off)
        sub.append((off, ln))
        off += ln

    mesh = plsc.VectorSubcoreMesh(core_axis_name="c", subcore_axis_name="s")

    @functools.partial(
        pl.kernel, mesh=mesh,
        out_type=jax.ShapeDtypeStruct((m_rows * KNN, c), jnp.float32),
        scratch_types=[
            pltpu.VMEM((nidx,), jnp.int32),
            pltpu.VMEM((nidx, c), jnp.float32),
            pltpu.VMEM((nidx, c), jnp.float32),
            pltpu.SemaphoreType.DMA,
            pltpu.SemaphoreType.DMA,
        ],
    )
    def k(x_hbm, idx_hbm, xg_hbm, idxb, rows0, rows1, gsem, wsem):
        wid = lax.axis_index("s") * nc + lax.axis_index("c")
        base = wid * rw
        boff = (base // n_pts) * n_pts

        def fetch(ci, rows):
            rbase = base + ci * chunk
            pltpu.sync_copy(idx_hbm.at[pl.ds(rbase * KNN, nidx)], idxb)
            for j in range(nidx // 16):
                sl = pl.ds(j * 16, 16)
                idxb[sl] = idxb[sl] + boff
            return [
                pltpu.async_copy(x_hbm.at[idxb.at[pl.ds(soff, slen)]],
                                 rows.at[pl.ds(soff, slen)], gsem)
                for soff, slen in sub
            ]

        nch = rw // chunk
        bufs = (rows0, rows1)
        hs = fetch(0, rows0)
        for ci in range(nch):
            for h in hs:
                h.wait()
            cur = bufs[ci % 2]
            if ci + 1 < nch:
                hs = fetch(ci + 1, bufs[(ci + 1) % 2])
            rbase = base + ci * chunk
            pltpu.async_copy(
                cur, xg_hbm.at[pl.ds(rbase * KNN, nidx)], wsem).wait()

    return k(xf, idxf)


def _edge_body(c_dim, w_ref, xg_ref, xi_ref, mx_ref, s1_ref, s2_ref):
    bi = pl.program_id(0)
    j = pl.program_id(1)
    w = w_ref[...]
    xi = xi_ref[0][:, :c_dim]
    mx = None
    a1 = None
    a2 = None
    for kk in range(KNN):
        xj = xg_ref[0][:, kk * CPAD:kk * CPAD + c_dim]
        f = jnp.concatenate([xj - xi, xi], axis=1)
        y = lax.dot_general(f, w, (((1,), (1,)), ((), ())),
                            preferred_element_type=jnp.float32)
        mx = y if mx is None else jnp.maximum(mx, y)
        a1 = y if a1 is None else a1 + y
        a2 = y * y if a2 is None else a2 + y * y
    mx_ref[0] = mx
    ls1 = jnp.sum(a1, axis=0)[None, :]
    ls2 = jnp.sum(a2, axis=0)[None, :]

    @pl.when((j == 0) & (bi == 0))
    def _():
        s1_ref[...] = ls1
        s2_ref[...] = ls2

    @pl.when((j != 0) | (bi != 0))
    def _():
        s1_ref[...] = s1_ref[...] + ls1
        s2_ref[...] = s2_ref[...] + ls2


def _edge(xg, x, w, c_dim, row_block=512):
    b, n, _ = x.shape
    o = w.shape[0]
    return pl.pallas_call(
        functools.partial(_edge_body, c_dim),
        grid=(b, n // row_block),
        in_specs=[
            pl.BlockSpec((o, 2 * c_dim), lambda bi, j: (0, 0)),
            pl.BlockSpec((1, row_block, KNN * CPAD), lambda bi, j: (bi, j, 0)),
            pl.BlockSpec((1, row_block, CPAD), lambda bi, j: (bi, j, 0)),
        ],
        out_specs=[
            pl.BlockSpec((1, row_block, o), lambda bi, j: (bi, j, 0)),
            pl.BlockSpec((1, o), lambda bi, j: (0, 0)),
            pl.BlockSpec((1, o), lambda bi, j: (0, 0)),
        ],
        out_shape=[
            jax.ShapeDtypeStruct((b, n, o), jnp.float32),
            jax.ShapeDtypeStruct((1, o), jnp.float32),
            jax.ShapeDtypeStruct((1, o), jnp.float32),
        ],
        compiler_params=pltpu.CompilerParams(
            dimension_semantics=("arbitrary", "arbitrary")),
    )(w, xg.reshape(b, n, KNN * CPAD), x)


def _affine_lrelu_body(o_dim, x_ref, mu_ref, sg_ref, g_ref, b_ref, o_ref):
    z = (x_ref[0] - mu_ref[...]) / sg_ref[...] * g_ref[...] + b_ref[...]
    z = jnp.where(z > 0, z, 0.2 * z)
    if o_dim < CPAD:
        z = jnp.concatenate(
            [z, jnp.zeros((z.shape[0], CPAD - o_dim), jnp.float32)], axis=1)
    o_ref[0] = z


def _affine_lrelu_pad(x, mean, sig, g, beta, row_block=512):
    b, n, o = x.shape
    vec = lambda v: v.reshape(1, o)
    return pl.pallas_call(
        functools.partial(_affine_lrelu_body, o),
        grid=(b, n // row_block),
        in_specs=[pl.BlockSpec((1, row_block, o), lambda bi, j: (bi, j, 0))]
        + [pl.BlockSpec((1, o), lambda bi, j: (0, 0))] * 4,
        out_specs=pl.BlockSpec((1, row_block, max(o, CPAD)),
                               lambda bi, j: (bi, j, 0)),
        out_shape=jax.ShapeDtypeStruct((b, n, max(o, CPAD)), jnp.float32),
        compiler_params=pltpu.CompilerParams(
            dimension_semantics=("arbitrary", "arbitrary")),
    )(x, vec(mean), vec(sig), vec(g), vec(beta))


def _head5_body(w5_ref, w6b_ref, cat_ref, y5max_ref, s15_ref, s25_ref,
                r6_ref, rs1_ref, rs2_ref):
    bi = pl.program_id(0)
    j = pl.program_id(1)
    cat = cat_ref[0]
    y5 = lax.dot_general(cat, w5_ref[...], (((1,), (1,)), ((), ())),
                         preferred_element_type=jnp.float32)
    r6 = lax.dot_general(cat, w6b_ref[...], (((1,), (1,)), ((), ())),
                         preferred_element_type=jnp.float32)
    r6_ref[0] = r6
    bm = jnp.max(y5, axis=0)[None, None, :]
    ls1 = jnp.sum(y5, axis=0)[None, :]
    ls2 = jnp.sum(y5 * y5, axis=0)[None, :]
    lr1 = jnp.sum(r6, axis=0)[None, None, :]
    lr2 = jnp.sum(r6 * r6, axis=0)[None, None, :]

    @pl.when(j == 0)
    def _():
        y5max_ref[...] = bm
        rs1_ref[...] = lr1
        rs2_ref[...] = lr2

    @pl.when(j != 0)
    def _():
        y5max_ref[...] = jnp.maximum(y5max_ref[...], bm)
        rs1_ref[...] = rs1_ref[...] + lr1
        rs2_ref[...] = rs2_ref[...] + lr2

    @pl.when((j == 0) & (bi == 0))
    def _():
        s15_ref[...] = ls1
        s25_ref[...] = ls2

    @pl.when((j != 0) | (bi != 0))
    def _():
        s15_ref[...] = s15_ref[...] + ls1
        s25_ref[...] = s25_ref[...] + ls2


def _head5(cat, w5, w6b, row_block=512):
    b, n, ci = cat.shape
    o5 = w5.shape[0]
    o6 = w6b.shape[0]
    return pl.pallas_call(
        _head5_body,
        grid=(b, n // row_block),
        in_specs=[
            pl.BlockSpec((o5, ci), lambda bi, j: (0, 0)),
            pl.BlockSpec((o6, ci), lambda bi, j: (0, 0)),
            pl.BlockSpec((1, row_block, ci), lambda bi, j: (bi, j, 0)),
        ],
        out_specs=[
            pl.BlockSpec((1, 1, o5), lambda bi, j: (bi, 0, 0)),
            pl.BlockSpec((1, o5), lambda bi, j: (0, 0)),
            pl.BlockSpec((1, o5), lambda bi, j: (0, 0)),
            pl.BlockSpec((1, row_block, o6), lambda bi, j: (bi, j, 0)),
            pl.BlockSpec((1, 1, o6), lambda bi, j: (bi, 0, 0)),
            pl.BlockSpec((1, 1, o6), lambda bi, j: (bi, 0, 0)),
        ],
        out_shape=[
            jax.ShapeDtypeStruct((b, 1, o5), jnp.float32),
            jax.ShapeDtypeStruct((1, o5), jnp.float32),
            jax.ShapeDtypeStruct((1, o5), jnp.float32),
            jax.ShapeDtypeStruct((b, n, o6), jnp.float32),
            jax.ShapeDtypeStruct((b, 1, o6), jnp.float32),
            jax.ShapeDtypeStruct((b, 1, o6), jnp.float32),
        ],
        compiler_params=pltpu.CompilerParams(
            dimension_semantics=("arbitrary", "arbitrary")),
    )(w5, w6b, cat)


def _bias6_body(g_ref, w_ref, o_ref):
    o_ref[...] = lax.dot_general(g_ref[...], w_ref[...],
                                 (((1,), (1,)), ((), ())),
                                 preferred_element_type=jnp.float32)


def _bias6(gmax, w6a):
    b, _ = gmax.shape
    o = w6a.shape[0]
    return pl.pallas_call(
        _bias6_body,
        out_shape=jax.ShapeDtypeStruct((b, o), jnp.float32),
    )(gmax, w6a)


def _head7_body(w7_ref, mu_ref, sg_ref, g_ref, be_ref, r6_ref, bias_ref,
                y7max_ref, s17_ref, s27_ref):
    bi = pl.program_id(0)
    j = pl.program_id(1)
    y6 = r6_ref[0] + bias_ref[0]
    z = (y6 - mu_ref[...]) / sg_ref[...] * g_ref[...] + be_ref[...]
    h = jnp.where(z > 0, z, 0.2 * z)
    y7 = lax.dot_general(h, w7_ref[...], (((1,), (1,)), ((), ())),
                         preferred_element_type=jnp.float32)
    bm = jnp.max(y7, axis=0)[None, None, :]
    ls1 = jnp.sum(y7, axis=0)[None, :]
    ls2 = jnp.sum(y7 * y7, axis=0)[None, :]

    @pl.when(j == 0)
    def _():
        y7max_ref[...] = bm

    @pl.when(j != 0)
    def _():
        y7max_ref[...] = jnp.maximum(y7max_ref[...], bm)

    @pl.when((j == 0) & (bi == 0))
    def _():
        s17_ref[...] = ls1
        s27_ref[...] = ls2

    @pl.when((j != 0) | (bi != 0))
    def _():
        s17_ref[...] = s17_ref[...] + ls1
        s27_ref[...] = s27_ref[...] + ls2


def _head7(r6, bias, mean, sig, g6, b6, w7, row_block=512):
    b, n, ci = r6.shape
    o = w7.shape[0]
    vec = lambda v: v.reshape(1, ci)
    return pl.pallas_call(
        _head7_body,
        grid=(b, n // row_block),
        in_specs=[pl.BlockSpec((o, ci), lambda bi, j: (0, 0))]
        + [pl.BlockSpec((1, ci), lambda bi, j: (0, 0))] * 4
        + [
            pl.BlockSpec((1, row_block, ci), lambda bi, j: (bi, j, 0)),
            pl.BlockSpec((1, 1, ci), lambda bi, j: (bi, 0, 0)),
        ],
        out_specs=[
            pl.BlockSpec((1, 1, o), lambda bi, j: (bi, 0, 0)),
            pl.BlockSpec((1, o), lambda bi, j: (0, 0)),
            pl.BlockSpec((1, o), lambda bi, j: (0, 0)),
        ],
        out_shape=[
            jax.ShapeDtypeStruct((b, 1, o), jnp.float32),
            jax.ShapeDtypeStruct((1, o), jnp.float32),
            jax.ShapeDtypeStruct((1, o), jnp.float32),
        ],
        compiler_params=pltpu.CompilerParams(
            dimension_semantics=("arbitrary", "arbitrary")),
    )(w7, vec(mean), vec(sig), vec(g6), vec(b6), r6,
      bias.reshape(b, 1, ci))


def _bn_stats(s1, s2, count):
    mean = s1 / count
    var = s2 / count - mean * mean
    return mean, jnp.sqrt(var + 1e-5)


def _lrelu(z):
    return jnp.where(z > 0, z, 0.2 * z)


def _edgeconv(x, c_dim, w, g, bb):
    b, n, _ = x.shape
    grp = 2
    mxs, s1s, s2s = [], [], []
    for bi in range(0, b, grp):
        xb = lax.slice_in_dim(x, bi, bi + grp, axis=0)
        idx = _knn(xb)
        xg = _sc_gather(xb.reshape(grp * n, CPAD),
                        idx.reshape(grp * n * KNN), n)
        mxb, s1b, s2b = _edge(xg, xb, w, c_dim)
        mxs.append(mxb)
        s1s.append(s1b)
        s2s.append(s2b)
    s1 = sum(v[0] for v in s1s)
    s2 = sum(v[0] for v in s2s)
    mx = jnp.concatenate(mxs, axis=0)
    mean, sig = _bn_stats(s1, s2, b * n * KNN)
    return _affine_lrelu_pad(mx, mean, sig, g, bb)


def kernel(x, W1, W2, W3, W4, W5, W6, W7,
           g1, g2, g3, g4, g5, g6, g7,
           b1, b2, b3, b4, b5, b6, b7):
    b, n, c0 = x.shape
    xp = jnp.pad(x, ((0, 0), (0, 0), (0, CPAD - c0)))
    x1 = _edgeconv(xp, c0, W1, g1, b1)
    x2 = _edgeconv(x1, 64, W2, g2, b2)
    x3 = _edgeconv(x2, 64, W3, g3, b3)
    x4 = _edgeconv(x3, 128, W4, g4, b4)
    cat = jnp.concatenate([x1[..., :64], x2[..., :64], x3, x4], axis=-1)

    w6a, w6b = W6[:, :W5.shape[0]], W6[:, W5.shape[0]:]
    y5max, s15, s25, r6, rs1, rs2 = _head5(cat, W5, w6b)
    y5max, rs1, rs2 = y5max[:, 0], rs1[:, 0], rs2[:, 0]
    p = b * n
    mean5, sig5 = _bn_stats(s15[0], s25[0], p)
    gmax = _lrelu((y5max - mean5) / sig5 * g5 + b5)
    bias = _bias6(gmax, w6a)

    mean6 = jnp.sum(rs1 + n * bias, axis=0) / p
    e2 = jnp.sum(rs2 + 2.0 * bias * rs1 + n * bias * bias, axis=0) / p
    var6 = e2 - mean6 * mean6
    sig6 = jnp.sqrt(var6 + 1e-5)

    y7max, s17, s27 = _head7(r6, bias, mean6, sig6, g6, b6, W7)
    mean7, sig7 = _bn_stats(s17[0], s27[0], p)
    return _lrelu((y7max[:, 0] - mean7) / sig7 * g7 + b7)


# speedup vs baseline: 1.2564x; 1.0404x over previous
import functools
import jax
import jax.numpy as jnp
from jax import lax
from jax.experimental import pallas as pl
from jax.experimental.pallas import tpu as pltpu
from jax.experimental.pallas import tpu_sc as plsc

KNN = 20
CPAD = 128
F32_MIN = jnp.finfo(jnp.float32).min


def _knn_body(n_pts, xr_ref, xa_ref, idx_ref):
    xr = xr_ref[0]
    xa = xa_ref[0]
    g = lax.dot_general(xr, xa, (((1,), (1,)), ((), ())),
                        preferred_element_type=jnp.float32)
    inner = -2.0 * g
    s = (-jnp.sum(xr * xr, axis=1)[:, None] - inner) \
        - jnp.sum(xa * xa, axis=1)[None, :]
    iota = lax.broadcasted_iota(jnp.int32, s.shape, 1)
    sels = []
    for _ in range(KNN):
        m = jnp.max(s, axis=1, keepdims=True)
        sel = jnp.min(jnp.where(s >= m, iota, n_pts), axis=1)
        sels.append(sel)
        s = jnp.where(iota == sel[:, None], F32_MIN, s)
    idx_ref[0] = jnp.stack(sels, axis=1)


def _knn(x, row_block=512):
    b, n, c = x.shape
    return pl.pallas_call(
        functools.partial(_knn_body, n),
        grid=(b, n // row_block),
        in_specs=[
            pl.BlockSpec((1, row_block, c), lambda bi, j: (bi, j, 0)),
            pl.BlockSpec((1, n, c), lambda bi, j: (bi, 0, 0)),
        ],
        out_specs=pl.BlockSpec((1, row_block, KNN), lambda bi, j: (bi, j, 0)),
        out_shape=jax.ShapeDtypeStruct((b, n, KNN), jnp.int32),
        compiler_params=pltpu.CompilerParams(
            dimension_semantics=("arbitrary", "arbitrary")),
    )(x, x)


def _sc_gather(xf, idxf, n_pts, chunk=16):
    m_rows, c = xf.shape
    info = plsc.get_sparse_core_info()
    nc, ns = info.num_cores, info.num_subcores
    nw = nc * ns
    rw = m_rows // nw
    assert rw % chunk == 0 and n_pts % rw == 0
    nidx = chunk * KNN
    sub = []
    off = 0
    while off < nidx:
        ln = min(128, nidx - off)
        sub.append((off, ln))
        off += ln

    mesh = plsc.VectorSubcoreMesh(core_axis_name="c", subcore_axis_name="s")

    @functools.partial(
        pl.kernel, mesh=mesh,
        out_type=jax.ShapeDtypeStruct((m_rows * KNN, c), jnp.float32),
        scratch_types=[
            pltpu.VMEM((nidx,), jnp.int32),
            pltpu.VMEM((nidx, c), jnp.float32),
            pltpu.VMEM((nidx, c), jnp.float32),
            pltpu.SemaphoreType.DMA,
            pltpu.SemaphoreType.DMA,
        ],
    )
    def k(x_hbm, idx_hbm, xg_hbm, idxb, rows0, rows1, gsem, wsem):
        wid = lax.axis_index("s") * nc + lax.axis_index("c")
        base = wid * rw
        boff = (base // n_pts) * n_pts

        def fetch(ci, rows):
            rbase = base + ci * chunk
            pltpu.sync_copy(idx_hbm.at[pl.ds(rbase * KNN, nidx)], idxb)
            for j in range(nidx // 16):
                sl = pl.ds(j * 16, 16)
                idxb[sl] = idxb[sl] + boff
            return [
                pltpu.async_copy(x_hbm.at[idxb.at[pl.ds(soff, slen)]],
                                 rows.at[pl.ds(soff, slen)], gsem)
                for soff, slen in sub
            ]

        nch = rw // chunk
        bufs = (rows0, rows1)
        hs = fetch(0, rows0)
        for ci in range(nch):
            for h in hs:
                h.wait()
            cur = bufs[ci % 2]
            if ci + 1 < nch:
                hs = fetch(ci + 1, bufs[(ci + 1) % 2])
            rbase = base + ci * chunk
            pltpu.async_copy(
                cur, xg_hbm.at[pl.ds(rbase * KNN, nidx)], wsem).wait()

    return k(xf, idxf)


def _edge_body(c_dim, w_ref, xg_ref, xi_ref, mx_ref, s1_ref, s2_ref):
    bi = pl.program_id(0)
    j = pl.program_id(1)
    w = w_ref[...]
    xi = xi_ref[0][:, :c_dim]
    mx = None
    a1 = None
    a2 = None
    for kk in range(KNN):
        xj = xg_ref[0][:, kk * CPAD:kk * CPAD + c_dim]
        f = jnp.concatenate([xj - xi, xi], axis=1)
        y = lax.dot_general(f, w, (((1,), (1,)), ((), ())),
                            preferred_element_type=jnp.float32)
        mx = y if mx is None else jnp.maximum(mx, y)
        a1 = y if a1 is None else a1 + y
        a2 = y * y if a2 is None else a2 + y * y
    mx_ref[0] = mx
    ls1 = jnp.sum(a1, axis=0)[None, :]
    ls2 = jnp.sum(a2, axis=0)[None, :]

    @pl.when((j == 0) & (bi == 0))
    def _():
        s1_ref[...] = ls1
        s2_ref[...] = ls2

    @pl.when((j != 0) | (bi != 0))
    def _():
        s1_ref[...] = s1_ref[...] + ls1
        s2_ref[...] = s2_ref[...] + ls2


def _edge(xg, x, w, c_dim, row_block=512):
    b, n, _ = x.shape
    o = w.shape[0]
    return pl.pallas_call(
        functools.partial(_edge_body, c_dim),
        grid=(b, n // row_block),
        in_specs=[
            pl.BlockSpec((o, 2 * c_dim), lambda bi, j: (0, 0)),
            pl.BlockSpec((1, row_block, KNN * CPAD), lambda bi, j: (bi, j, 0)),
            pl.BlockSpec((1, row_block, CPAD), lambda bi, j: (bi, j, 0)),
        ],
        out_specs=[
            pl.BlockSpec((1, row_block, o), lambda bi, j: (bi, j, 0)),
            pl.BlockSpec((1, o), lambda bi, j: (0, 0)),
            pl.BlockSpec((1, o), lambda bi, j: (0, 0)),
        ],
        out_shape=[
            jax.ShapeDtypeStruct((b, n, o), jnp.float32),
            jax.ShapeDtypeStruct((1, o), jnp.float32),
            jax.ShapeDtypeStruct((1, o), jnp.float32),
        ],
        compiler_params=pltpu.CompilerParams(
            dimension_semantics=("arbitrary", "arbitrary")),
    )(w, xg.reshape(b, n, KNN * CPAD), x)


def _affine_lrelu_body(o_dim, x_ref, mu_ref, sg_ref, g_ref, b_ref, o_ref):
    z = (x_ref[0] - mu_ref[...]) / sg_ref[...] * g_ref[...] + b_ref[...]
    z = jnp.where(z > 0, z, 0.2 * z)
    if o_dim < CPAD:
        z = jnp.concatenate(
            [z, jnp.zeros((z.shape[0], CPAD - o_dim), jnp.float32)], axis=1)
    o_ref[0] = z


def _affine_lrelu_pad(x, mean, sig, g, beta, row_block=512):
    b, n, o = x.shape
    vec = lambda v: v.reshape(1, o)
    return pl.pallas_call(
        functools.partial(_affine_lrelu_body, o),
        grid=(b, n // row_block),
        in_specs=[pl.BlockSpec((1, row_block, o), lambda bi, j: (bi, j, 0))]
        + [pl.BlockSpec((1, o), lambda bi, j: (0, 0))] * 4,
        out_specs=pl.BlockSpec((1, row_block, max(o, CPAD)),
                               lambda bi, j: (bi, j, 0)),
        out_shape=jax.ShapeDtypeStruct((b, n, max(o, CPAD)), jnp.float32),
        compiler_params=pltpu.CompilerParams(
            dimension_semantics=("arbitrary", "arbitrary")),
    )(x, vec(mean), vec(sig), vec(g), vec(beta))


def _head5_body(w5_ref, w6b_ref, cat_ref, y5max_ref, s15_ref, s25_ref,
                r6_ref, rs1_ref, rs2_ref):
    bi = pl.program_id(0)
    j = pl.program_id(1)
    cat = cat_ref[0]
    y5 = lax.dot_general(cat, w5_ref[...], (((1,), (1,)), ((), ())),
                         preferred_element_type=jnp.float32)
    r6 = lax.dot_general(cat, w6b_ref[...], (((1,), (1,)), ((), ())),
                         preferred_element_type=jnp.float32)
    r6_ref[0] = r6
    bm = jnp.max(y5, axis=0)[None, None, :]
    ls1 = jnp.sum(y5, axis=0)[None, :]
    ls2 = jnp.sum(y5 * y5, axis=0)[None, :]
    lr1 = jnp.sum(r6, axis=0)[None, None, :]
    lr2 = jnp.sum(r6 * r6, axis=0)[None, None, :]

    @pl.when(j == 0)
    def _():
        y5max_ref[...] = bm
        rs1_ref[...] = lr1
        rs2_ref[...] = lr2

    @pl.when(j != 0)
    def _():
        y5max_ref[...] = jnp.maximum(y5max_ref[...], bm)
        rs1_ref[...] = rs1_ref[...] + lr1
        rs2_ref[...] = rs2_ref[...] + lr2

    @pl.when((j == 0) & (bi == 0))
    def _():
        s15_ref[...] = ls1
        s25_ref[...] = ls2

    @pl.when((j != 0) | (bi != 0))
    def _():
        s15_ref[...] = s15_ref[...] + ls1
        s25_ref[...] = s25_ref[...] + ls2


def _head5(cat, w5, w6b, row_block=512):
    b, n, ci = cat.shape
    o5 = w5.shape[0]
    o6 = w6b.shape[0]
    return pl.pallas_call(
        _head5_body,
        grid=(b, n // row_block),
        in_specs=[
            pl.BlockSpec((o5, ci), lambda bi, j: (0, 0)),
            pl.BlockSpec((o6, ci), lambda bi, j: (0, 0)),
            pl.BlockSpec((1, row_block, ci), lambda bi, j: (bi, j, 0)),
        ],
        out_specs=[
            pl.BlockSpec((1, 1, o5), lambda bi, j: (bi, 0, 0)),
            pl.BlockSpec((1, o5), lambda bi, j: (0, 0)),
            pl.BlockSpec((1, o5), lambda bi, j: (0, 0)),
            pl.BlockSpec((1, row_block, o6), lambda bi, j: (bi, j, 0)),
            pl.BlockSpec((1, 1, o6), lambda bi, j: (bi, 0, 0)),
            pl.BlockSpec((1, 1, o6), lambda bi, j: (bi, 0, 0)),
        ],
        out_shape=[
            jax.ShapeDtypeStruct((b, 1, o5), jnp.float32),
            jax.ShapeDtypeStruct((1, o5), jnp.float32),
            jax.ShapeDtypeStruct((1, o5), jnp.float32),
            jax.ShapeDtypeStruct((b, n, o6), jnp.float32),
            jax.ShapeDtypeStruct((b, 1, o6), jnp.float32),
            jax.ShapeDtypeStruct((b, 1, o6), jnp.float32),
        ],
        compiler_params=pltpu.CompilerParams(
            dimension_semantics=("arbitrary", "arbitrary")),
    )(w5, w6b, cat)


def _bias6_body(g_ref, w_ref, o_ref):
    o_ref[...] = lax.dot_general(g_ref[...], w_ref[...],
                                 (((1,), (1,)), ((), ())),
                                 preferred_element_type=jnp.float32)


def _bias6(gmax, w6a):
    b, _ = gmax.shape
    o = w6a.shape[0]
    return pl.pallas_call(
        _bias6_body,
        out_shape=jax.ShapeDtypeStruct((b, o), jnp.float32),
    )(gmax, w6a)


def _head7_body(w7_ref, mu_ref, sg_ref, g_ref, be_ref, r6_ref, bias_ref,
                y7max_ref, s17_ref, s27_ref):
    bi = pl.program_id(0)
    j = pl.program_id(1)
    y6 = r6_ref[0] + bias_ref[0]
    z = (y6 - mu_ref[...]) / sg_ref[...] * g_ref[...] + be_ref[...]
    h = jnp.where(z > 0, z, 0.2 * z)
    y7 = lax.dot_general(h, w7_ref[...], (((1,), (1,)), ((), ())),
                         preferred_element_type=jnp.float32)
    bm = jnp.max(y7, axis=0)[None, None, :]
    ls1 = jnp.sum(y7, axis=0)[None, :]
    ls2 = jnp.sum(y7 * y7, axis=0)[None, :]

    @pl.when(j == 0)
    def _():
        y7max_ref[...] = bm

    @pl.when(j != 0)
    def _():
        y7max_ref[...] = jnp.maximum(y7max_ref[...], bm)

    @pl.when((j == 0) & (bi == 0))
    def _():
        s17_ref[...] = ls1
        s27_ref[...] = ls2

    @pl.when((j != 0) | (bi != 0))
    def _():
        s17_ref[...] = s17_ref[...] + ls1
        s27_ref[...] = s27_ref[...] + ls2


def _head7(r6, bias, mean, sig, g6, b6, w7, row_block=512):
    b, n, ci = r6.shape
    o = w7.shape[0]
    vec = lambda v: v.reshape(1, ci)
    return pl.pallas_call(
        _head7_body,
        grid=(b, n // row_block),
        in_specs=[pl.BlockSpec((o, ci), lambda bi, j: (0, 0))]
        + [pl.BlockSpec((1, ci), lambda bi, j: (0, 0))] * 4
        + [
            pl.BlockSpec((1, row_block, ci), lambda bi, j: (bi, j, 0)),
            pl.BlockSpec((1, 1, ci), lambda bi, j: (bi, 0, 0)),
        ],
        out_specs=[
            pl.BlockSpec((1, 1, o), lambda bi, j: (bi, 0, 0)),
            pl.BlockSpec((1, o), lambda bi, j: (0, 0)),
            pl.BlockSpec((1, o), lambda bi, j: (0, 0)),
        ],
        out_shape=[
            jax.ShapeDtypeStruct((b, 1, o), jnp.float32),
            jax.ShapeDtypeStruct((1, o), jnp.float32),
            jax.ShapeDtypeStruct((1, o), jnp.float32),
        ],
        compiler_params=pltpu.CompilerParams(
            dimension_semantics=("arbitrary", "arbitrary")),
    )(w7, vec(mean), vec(sig), vec(g6), vec(b6), r6,
      bias.reshape(b, 1, ci))


def _bn_stats(s1, s2, count):
    mean = s1 / count
    var = s2 / count - mean * mean
    return mean, jnp.sqrt(var + 1e-5)


def _lrelu(z):
    return jnp.where(z > 0, z, 0.2 * z)


def _edgeconv(x, c_dim, w, g, bb):
    b, n, _ = x.shape
    grp = 4
    mxs, s1s, s2s = [], [], []
    for bi in range(0, b, grp):
        xb = lax.slice_in_dim(x, bi, bi + grp, axis=0)
        idx = _knn(xb)
        xg = _sc_gather(xb.reshape(grp * n, CPAD),
                        idx.reshape(grp * n * KNN), n)
        mxb, s1b, s2b = _edge(xg, xb, w, c_dim)
        mxs.append(mxb)
        s1s.append(s1b)
        s2s.append(s2b)
    s1 = sum(v[0] for v in s1s)
    s2 = sum(v[0] for v in s2s)
    mx = jnp.concatenate(mxs, axis=0)
    mean, sig = _bn_stats(s1, s2, b * n * KNN)
    return _affine_lrelu_pad(mx, mean, sig, g, bb)


def kernel(x, W1, W2, W3, W4, W5, W6, W7,
           g1, g2, g3, g4, g5, g6, g7,
           b1, b2, b3, b4, b5, b6, b7):
    b, n, c0 = x.shape
    xp = jnp.pad(x, ((0, 0), (0, 0), (0, CPAD - c0)))
    x1 = _edgeconv(xp, c0, W1, g1, b1)
    x2 = _edgeconv(x1, 64, W2, g2, b2)
    x3 = _edgeconv(x2, 64, W3, g3, b3)
    x4 = _edgeconv(x3, 128, W4, g4, b4)
    cat = jnp.concatenate([x1[..., :64], x2[..., :64], x3, x4], axis=-1)

    w6a, w6b = W6[:, :W5.shape[0]], W6[:, W5.shape[0]:]
    y5max, s15, s25, r6, rs1, rs2 = _head5(cat, W5, w6b)
    y5max, rs1, rs2 = y5max[:, 0], rs1[:, 0], rs2[:, 0]
    p = b * n
    mean5, sig5 = _bn_stats(s15[0], s25[0], p)
    gmax = _lrelu((y5max - mean5) / sig5 * g5 + b5)
    bias = _bias6(gmax, w6a)

    mean6 = jnp.sum(rs1 + n * bias, axis=0) / p
    e2 = jnp.sum(rs2 + 2.0 * bias * rs1 + n * bias * bias, axis=0) / p
    var6 = e2 - mean6 * mean6
    sig6 = jnp.sqrt(var6 + 1e-5)

    y7max, s17, s27 = _head7(r6, bias, mean6, sig6, g6, b6, W7)
    mean7, sig7 = _bn_stats(s17[0], s27[0], p)
    return _lrelu((y7max[:, 0] - mean7) / sig7 * g7 + b7)
